# baseline (speedup 1.0000x reference)
import functools

import jax
import jax.numpy as jnp
from jax import lax
from jax.experimental import pallas as pl
from jax.experimental.pallas import tpu as pltpu
from jax.experimental.pallas import tpu_sc as plsc

D = 128
NTILES = 32
NSUB = 16
NPAD = 10240
ROWS_PER_TILE = NPAD // NSUB

CHUNK = 128
E2PAD = 655360
TOTAL_CHUNKS = E2PAD // CHUNK
IDX_ROWS = TOTAL_CHUNKS
GRP = 16
NBUF = 2
CPT0 = 320
CPT1 = (TOTAL_CHUNKS - NSUB * CPT0) // NSUB

HCHUNK = 128
HCPT = E2PAD // (NTILES * HCHUNK)
HROWS = E2PAD // HCHUNK


def _mesh():
    return plsc.VectorSubcoreMesh(core_axis_name="c", subcore_axis_name="s")


@functools.partial(
    pl.kernel,
    mesh=_mesh(),
    out_type=jax.ShapeDtypeStruct((2, NPAD), jnp.float32),
    scratch_types=[
        pltpu.VMEM((HCPT, HCHUNK), jnp.int32),
        pltpu.VMEM((HCHUNK,), jnp.float32),
        pltpu.VMEM_SHARED((NPAD,), jnp.float32),
    ],
)
def _degree_kernel(sidx_hbm, zeros_hbm, deg_hbm, idx_v, ones_v, acc_sh):
    core = lax.axis_index("c")
    sid = lax.axis_index("s")
    wid = core * NSUB + sid

    @pl.loop(0, HCHUNK // 16)
    def _(i):
        ones_v[pl.ds(i * 16, 16)] = jnp.full((16,), 1.0, jnp.float32)

    pltpu.sync_copy(zeros_hbm, acc_sh.at[pl.ds(sid * ROWS_PER_TILE, ROWS_PER_TILE)])
    pltpu.sync_copy(sidx_hbm.at[pl.ds(wid * HCPT, HCPT)], idx_v)
    plsc.subcore_barrier()

    @pl.loop(0, HCPT)
    def _(j):
        pltpu.sync_copy(ones_v, acc_sh.at[idx_v.at[j]], add=True)

    plsc.subcore_barrier()
    pltpu.sync_copy(
        acc_sh.at[pl.ds(sid * ROWS_PER_TILE, ROWS_PER_TILE)],
        deg_hbm.at[core, pl.ds(sid * ROWS_PER_TILE, ROWS_PER_TILE)])


def _agg_loop(y_hbm, sidx_hbm, didx_hbm, acc_sh, sidx_v, didx_v,
              rows0, rows1, gsem0, gsem1, ssem0, ssem1, base0, n_groups):
    @pl.loop(0, n_groups)
    def _(g):
        base = base0 + g * GRP
        pltpu.sync_copy(sidx_hbm.at[pl.ds(base, GRP)], sidx_v)
        pltpu.sync_copy(didx_hbm.at[pl.ds(base, GRP)], didx_v)
        pltpu.async_copy(y_hbm.at[sidx_v.at[0]], rows0, gsem0)
        pltpu.async_copy(y_hbm.at[sidx_v.at[1]], rows1, gsem1)

        @pl.loop(0, GRP // 2)
        def _(p):
            j = 2 * p
            pltpu.make_async_copy(y_hbm.at[sidx_v.at[j]], rows0, gsem0).wait()
            pltpu.async_copy(rows0, acc_sh.at[didx_v.at[j]], ssem0, add=True)
            pltpu.make_async_copy(y_hbm.at[sidx_v.at[j + 1]], rows1, gsem1).wait()
            pltpu.async_copy(rows1, acc_sh.at[didx_v.at[j + 1]], ssem1, add=True)

            @pl.when(p < GRP // 2 - 1)
            def _():
                pltpu.make_async_copy(
                    rows0, acc_sh.at[didx_v.at[j]], ssem0).wait()
                pltpu.async_copy(y_hbm.at[sidx_v.at[j + 2]], rows0, gsem0)
                pltpu.make_async_copy(
                    rows1, acc_sh.at[didx_v.at[j + 1]], ssem1).wait()
                pltpu.async_copy(y_hbm.at[sidx_v.at[j + 3]], rows1, gsem1)

            @pl.when(p == GRP // 2 - 1)
            def _():
                pltpu.make_async_copy(
                    rows0, acc_sh.at[didx_v.at[j]], ssem0).wait()
                pltpu.make_async_copy(
                    rows1, acc_sh.at[didx_v.at[j + 1]], ssem1).wait()


@functools.partial(
    pl.kernel,
    mesh=_mesh(),
    out_type=jax.ShapeDtypeStruct((2, NPAD, D), jnp.float32),
    scratch_types=[
        pltpu.VMEM((GRP, CHUNK), jnp.int32),
        pltpu.VMEM((GRP, CHUNK), jnp.int32),
        pltpu.VMEM_SHARED((NPAD, D), jnp.float32),
        pltpu.SemaphoreType.DMA,
        pltpu.SemaphoreType.DMA,
        pltpu.SemaphoreType.DMA,
        pltpu.SemaphoreType.DMA,
    ],
)
def _aggregate_kernel(y_hbm, sidx_hbm, didx_hbm, zeros_hbm, z_hbm,
                      sidx_v, didx_v, acc_sh, gsem0, gsem1, ssem0, ssem1):
    pl.run_scoped(
        functools.partial(_aggregate_body, y_hbm, sidx_hbm, didx_hbm,
                          zeros_hbm, z_hbm, sidx_v, didx_v, acc_sh,
                          gsem0, gsem1, ssem0, ssem1),
        pltpu.VMEM((CHUNK, D), jnp.float32),
        pltpu.VMEM((CHUNK, D), jnp.float32))


def _aggregate_body(y_hbm, sidx_hbm, didx_hbm, zeros_hbm, z_hbm,
                    sidx_v, didx_v, acc_sh, gsem0, gsem1, ssem0, ssem1,
                    rows0, rows1):
    core = lax.axis_index("c")
    sid = lax.axis_index("s")

    pltpu.sync_copy(zeros_hbm, acc_sh.at[pl.ds(sid * ROWS_PER_TILE, ROWS_PER_TILE)])
    plsc.subcore_barrier()

    args = (y_hbm, sidx_hbm, didx_hbm, acc_sh, sidx_v, didx_v,
            rows0, rows1, gsem0, gsem1, ssem0, ssem1)

    @pl.when(core == 0)
    def _():
        _agg_loop(*args, base0=sid * CPT0, n_groups=CPT0 // GRP)

    @pl.when(core == 1)
    def _():
        _agg_loop(*args, base0=NSUB * CPT0 + sid * CPT1, n_groups=CPT1 // GRP)

    plsc.subcore_barrier()
    pltpu.sync_copy(
        acc_sh.at[pl.ds(sid * ROWS_PER_TILE, ROWS_PER_TILE)],
        z_hbm.at[core, pl.ds(sid * ROWS_PER_TILE, ROWS_PER_TILE)])


def _scale_body(deg2_ref, xp_ref, y_ref, dinv_ref):
    deg = deg2_ref[0, :] + deg2_ref[1, :]
    dinv = jnp.where(deg > 0.0, lax.rsqrt(jnp.maximum(deg, 1.0)), 0.0)
    dinv_ref[...] = dinv[:, None]
    y_ref[...] = xp_ref[...] * dinv[:, None]


def _final_body(x_ref, z_ref, dinv_ref, w_ref, b_ref, o_ref):
    zsum = z_ref[0] + z_ref[1]
    xz = x_ref[...] + dinv_ref[...] * zsum
    r = lax.dot_general(
        xz, w_ref[...], (((1,), (1,)), ((), ())),
        preferred_element_type=jnp.float32, precision=lax.Precision.HIGHEST)
    o_ref[...] = jnp.maximum(r + b_ref[...][None, :], 0.0)


def kernel(x, edge_index, W, b):
    n = x.shape[0]
    n_edges = edge_index.shape[1]
    src = edge_index[0].astype(jnp.int32)
    dst = edge_index[1].astype(jnp.int32)
    pad = jnp.full((E2PAD - 2 * n_edges,), n, jnp.int32)
    sidx = jnp.concatenate([src, dst, pad])
    didx = jnp.concatenate([dst, src, pad])
    sidx_h = sidx.reshape(HROWS, HCHUNK)
    sidx_a = sidx.reshape(IDX_ROWS, CHUNK)
    didx_a = didx.reshape(IDX_ROWS, CHUNK)
    xp = jnp.pad(x, ((0, NPAD - n), (0, 0)))
    zeros1 = jnp.zeros((ROWS_PER_TILE,), jnp.float32)
    zeros2 = jnp.zeros((ROWS_PER_TILE, D), jnp.float32)

    deg2 = _degree_kernel(sidx_h, zeros1)

    y, dinv = pl.pallas_call(
        _scale_body,
        out_shape=[
            jax.ShapeDtypeStruct((NPAD, D), jnp.float32),
            jax.ShapeDtypeStruct((NPAD, 1), jnp.float32),
        ],
    )(deg2, xp)

    z2 = _aggregate_kernel(y, sidx_a, didx_a, zeros2)

    blk = 1000
    out = pl.pallas_call(
        _final_body,
        grid=(n // blk,),
        in_specs=[
            pl.BlockSpec((blk, D), lambda i: (i, 0)),
            pl.BlockSpec((2, blk, D), lambda i: (0, i, 0)),
            pl.BlockSpec((blk, 1), lambda i: (i, 0)),
            pl.BlockSpec((D, D), lambda i: (0, 0)),
            pl.BlockSpec((D,), lambda i: (0,)),
        ],
        out_specs=pl.BlockSpec((blk, D), lambda i: (i, 0)),
        out_shape=jax.ShapeDtypeStruct((n, D), jnp.float32),
    )(x, z2, dinv, W, b)
    return out


# speedup vs baseline: 1.3650x; 1.3650x over previous
import functools

import jax
import jax.numpy as jnp
from jax import lax
from jax.experimental import pallas as pl
from jax.experimental.pallas import tpu as pltpu
from jax.experimental.pallas import tpu_sc as plsc

D = 128
NTILES = 32
NSUB = 16
NPAD = 10240
ROWS_PER_TILE = NPAD // NSUB

CHUNK = 128
E2PAD = 655360
TOTAL_CHUNKS = E2PAD // CHUNK
IDX_ROWS = TOTAL_CHUNKS
GRP = 16
NBUF = 2
CPT0 = 288
CPT1 = (TOTAL_CHUNKS - NSUB * CPT0) // NSUB

HCHUNK = 128
HCPT = E2PAD // (NTILES * HCHUNK)
HROWS = E2PAD // HCHUNK


def _mesh():
    return plsc.VectorSubcoreMesh(core_axis_name="c", subcore_axis_name="s")


@functools.partial(
    pl.kernel,
    mesh=_mesh(),
    out_type=jax.ShapeDtypeStruct((2, NPAD), jnp.float32),
    scratch_types=[
        pltpu.VMEM((HCPT, HCHUNK), jnp.int32),
        pltpu.VMEM((HCHUNK,), jnp.float32),
        pltpu.VMEM_SHARED((NPAD,), jnp.float32),
    ],
)
def _degree_kernel(sidx_hbm, zeros_hbm, deg_hbm, idx_v, ones_v, acc_sh):
    core = lax.axis_index("c")
    sid = lax.axis_index("s")
    wid = core * NSUB + sid

    @pl.loop(0, HCHUNK // 16)
    def _(i):
        ones_v[pl.ds(i * 16, 16)] = jnp.full((16,), 1.0, jnp.float32)

    pltpu.sync_copy(zeros_hbm, acc_sh.at[pl.ds(sid * ROWS_PER_TILE, ROWS_PER_TILE)])
    pltpu.sync_copy(sidx_hbm.at[pl.ds(wid * HCPT, HCPT)], idx_v)
    plsc.subcore_barrier()

    @pl.loop(0, HCPT)
    def _(j):
        pltpu.sync_copy(ones_v, acc_sh.at[idx_v.at[j]], add=True)

    plsc.subcore_barrier()
    pltpu.sync_copy(
        acc_sh.at[pl.ds(sid * ROWS_PER_TILE, ROWS_PER_TILE)],
        deg_hbm.at[core, pl.ds(sid * ROWS_PER_TILE, ROWS_PER_TILE)])


def _agg_loop(y_hbm, sidx_hbm, didx_hbm, acc_sh, sidx_v, didx_v,
              rows0, rows1, gsem0, gsem1, ssem0, ssem1, base0, n_groups):
    @pl.loop(0, n_groups)
    def _(g):
        base = base0 + g * GRP
        pltpu.sync_copy(sidx_hbm.at[pl.ds(base, GRP)], sidx_v)
        pltpu.sync_copy(didx_hbm.at[pl.ds(base, GRP)], didx_v)
        pltpu.async_copy(y_hbm.at[sidx_v.at[0]], rows0, gsem0)
        pltpu.async_copy(y_hbm.at[sidx_v.at[1]], rows1, gsem1)

        @pl.loop(0, GRP // 2)
        def _(p):
            j = 2 * p
            pltpu.make_async_copy(y_hbm.at[sidx_v.at[j]], rows0, gsem0).wait()
            pltpu.async_copy(rows0, acc_sh.at[didx_v.at[j]], ssem0, add=True)
            pltpu.make_async_copy(y_hbm.at[sidx_v.at[j + 1]], rows1, gsem1).wait()
            pltpu.async_copy(rows1, acc_sh.at[didx_v.at[j + 1]], ssem1, add=True)

            @pl.when(p < GRP // 2 - 1)
            def _():
                pltpu.make_async_copy(
                    rows0, acc_sh.at[didx_v.at[j]], ssem0).wait()
                pltpu.async_copy(y_hbm.at[sidx_v.at[j + 2]], rows0, gsem0)
                pltpu.make_async_copy(
                    rows1, acc_sh.at[didx_v.at[j + 1]], ssem1).wait()
                pltpu.async_copy(y_hbm.at[sidx_v.at[j + 3]], rows1, gsem1)

            @pl.when(p == GRP // 2 - 1)
            def _():
                pltpu.make_async_copy(
                    rows0, acc_sh.at[didx_v.at[j]], ssem0).wait()
                pltpu.make_async_copy(
                    rows1, acc_sh.at[didx_v.at[j + 1]], ssem1).wait()


@functools.partial(
    pl.kernel,
    mesh=_mesh(),
    out_type=jax.ShapeDtypeStruct((2, NPAD, D), jnp.float32),
    scratch_types=[
        pltpu.VMEM((GRP, CHUNK), jnp.int32),
        pltpu.VMEM((GRP, CHUNK), jnp.int32),
        pltpu.VMEM_SHARED((NPAD, D), jnp.float32),
        pltpu.SemaphoreType.DMA,
        pltpu.SemaphoreType.DMA,
        pltpu.SemaphoreType.DMA,
        pltpu.SemaphoreType.DMA,
    ],
)
def _aggregate_kernel(y_hbm, sidx_hbm, didx_hbm, zeros_hbm, z_hbm,
                      sidx_v, didx_v, acc_sh, gsem0, gsem1, ssem0, ssem1):
    pl.run_scoped(
        functools.partial(_aggregate_body, y_hbm, sidx_hbm, didx_hbm,
                          zeros_hbm, z_hbm, sidx_v, didx_v, acc_sh,
                          gsem0, gsem1, ssem0, ssem1),
        pltpu.VMEM((CHUNK, D), jnp.float32),
        pltpu.VMEM((CHUNK, D), jnp.float32))


def _aggregate_body(y_hbm, sidx_hbm, didx_hbm, zeros_hbm, z_hbm,
                    sidx_v, didx_v, acc_sh, gsem0, gsem1, ssem0, ssem1,
                    rows0, rows1):
    core = lax.axis_index("c")
    sid = lax.axis_index("s")

    pltpu.sync_copy(zeros_hbm, acc_sh.at[pl.ds(sid * ROWS_PER_TILE, ROWS_PER_TILE)])
    plsc.subcore_barrier()

    args = (y_hbm, sidx_hbm, didx_hbm, acc_sh, sidx_v, didx_v,
            rows0, rows1, gsem0, gsem1, ssem0, ssem1)

    @pl.when(core == 0)
    def _():
        _agg_loop(*args, base0=sid * CPT0, n_groups=CPT0 // GRP)

    @pl.when(core == 1)
    def _():
        _agg_loop(*args, base0=NSUB * CPT0 + sid * CPT1, n_groups=CPT1 // GRP)

    plsc.subcore_barrier()
    pltpu.sync_copy(
        acc_sh.at[pl.ds(sid * ROWS_PER_TILE, ROWS_PER_TILE)],
        z_hbm.at[core, pl.ds(sid * ROWS_PER_TILE, ROWS_PER_TILE)])


def _scale_body(deg2_ref, xp_ref, y_ref, dinv_ref):
    deg = deg2_ref[0, :] + deg2_ref[1, :]
    dinv = jnp.where(deg > 0.0, lax.rsqrt(jnp.maximum(deg, 1.0)), 0.0)
    dinv_ref[...] = dinv[:, None]
    y_ref[...] = xp_ref[...] * dinv[:, None]


def _final_body(x_ref, z_ref, dinv_ref, w_ref, b_ref, o_ref):
    zsum = z_ref[0] + z_ref[1]
    xz = x_ref[...] + dinv_ref[...] * zsum
    r = lax.dot_general(
        xz, w_ref[...], (((1,), (1,)), ((), ())),
        preferred_element_type=jnp.float32, precision=lax.Precision.HIGHEST)
    o_ref[...] = jnp.maximum(r + b_ref[...][None, :], 0.0)


def kernel(x, edge_index, W, b):
    n = x.shape[0]
    n_edges = edge_index.shape[1]
    src = edge_index[0].astype(jnp.int32)
    dst = edge_index[1].astype(jnp.int32)
    pad = jnp.full((E2PAD - 2 * n_edges,), n, jnp.int32)
    sidx = jnp.concatenate([src, dst, pad])
    didx = jnp.concatenate([dst, src, pad])
    sidx_h = sidx.reshape(HROWS, HCHUNK)
    sidx_a = sidx.reshape(IDX_ROWS, CHUNK)
    didx_a = didx.reshape(IDX_ROWS, CHUNK)
    xp = jnp.pad(x, ((0, NPAD - n), (0, 0)))
    zeros1 = jnp.zeros((ROWS_PER_TILE,), jnp.float32)
    zeros2 = jnp.zeros((ROWS_PER_TILE, D), jnp.float32)

    deg2 = _degree_kernel(sidx_h, zeros1)

    y, dinv = pl.pallas_call(
        _scale_body,
        out_shape=[
            jax.ShapeDtypeStruct((NPAD, D), jnp.float32),
            jax.ShapeDtypeStruct((NPAD, 1), jnp.float32),
        ],
    )(deg2, xp)

    z2 = _aggregate_kernel(y, sidx_a, didx_a, zeros2)

    blk = 1000
    out = pl.pallas_call(
        _final_body,
        grid=(n // blk,),
        in_specs=[
            pl.BlockSpec((blk, D), lambda i: (i, 0)),
            pl.BlockSpec((2, blk, D), lambda i: (0, i, 0)),
            pl.BlockSpec((blk, 1), lambda i: (i, 0)),
            pl.BlockSpec((D, D), lambda i: (0, 0)),
            pl.BlockSpec((D,), lambda i: (0,)),
        ],
        out_specs=pl.BlockSpec((blk, D), lambda i: (i, 0)),
        out_shape=jax.ShapeDtypeStruct((n, D), jnp.float32),
    )(x, z2, dinv, W, b)
    return out


# speedup vs baseline: 1.3663x; 1.0010x over previous
import functools

import jax
import jax.numpy as jnp
from jax import lax
from jax.experimental import pallas as pl
from jax.experimental.pallas import tpu as pltpu
from jax.experimental.pallas import tpu_sc as plsc

D = 128
NTILES = 32
NSUB = 16
NPAD = 10240
ROWS_PER_TILE = NPAD // NSUB

CHUNK = 128
E2PAD = 655360
TOTAL_CHUNKS = E2PAD // CHUNK
IDX_ROWS = TOTAL_CHUNKS
GRP = 16
NBUF = 2
CPT0 = 304
CPT1 = (TOTAL_CHUNKS - NSUB * CPT0) // NSUB

HCHUNK = 128
HCPT = E2PAD // (NTILES * HCHUNK)
HROWS = E2PAD // HCHUNK


def _mesh():
    return plsc.VectorSubcoreMesh(core_axis_name="c", subcore_axis_name="s")


@functools.partial(
    pl.kernel,
    mesh=_mesh(),
    out_type=jax.ShapeDtypeStruct((2, NPAD), jnp.float32),
    scratch_types=[
        pltpu.VMEM((HCPT, HCHUNK), jnp.int32),
        pltpu.VMEM((HCHUNK,), jnp.float32),
        pltpu.VMEM_SHARED((NPAD,), jnp.float32),
    ],
)
def _degree_kernel(sidx_hbm, zeros_hbm, deg_hbm, idx_v, ones_v, acc_sh):
    core = lax.axis_index("c")
    sid = lax.axis_index("s")
    wid = core * NSUB + sid

    @pl.loop(0, HCHUNK // 16)
    def _(i):
        ones_v[pl.ds(i * 16, 16)] = jnp.full((16,), 1.0, jnp.float32)

    pltpu.sync_copy(zeros_hbm, acc_sh.at[pl.ds(sid * ROWS_PER_TILE, ROWS_PER_TILE)])
    pltpu.sync_copy(sidx_hbm.at[pl.ds(wid * HCPT, HCPT)], idx_v)
    plsc.subcore_barrier()

    @pl.loop(0, HCPT)
    def _(j):
        pltpu.sync_copy(ones_v, acc_sh.at[idx_v.at[j]], add=True)

    plsc.subcore_barrier()
    pltpu.sync_copy(
        acc_sh.at[pl.ds(sid * ROWS_PER_TILE, ROWS_PER_TILE)],
        deg_hbm.at[core, pl.ds(sid * ROWS_PER_TILE, ROWS_PER_TILE)])


def _agg_loop(y_hbm, sidx_hbm, didx_hbm, acc_sh, sidx_v, didx_v,
              rows0, rows1, gsem0, gsem1, ssem0, ssem1, base0, n_groups):
    @pl.loop(0, n_groups)
    def _(g):
        base = base0 + g * GRP
        pltpu.sync_copy(sidx_hbm.at[pl.ds(base, GRP)], sidx_v)
        pltpu.sync_copy(didx_hbm.at[pl.ds(base, GRP)], didx_v)
        pltpu.async_copy(y_hbm.at[sidx_v.at[0]], rows0, gsem0)
        pltpu.async_copy(y_hbm.at[sidx_v.at[1]], rows1, gsem1)

        @pl.loop(0, GRP // 2)
        def _(p):
            j = 2 * p
            pltpu.make_async_copy(y_hbm.at[sidx_v.at[j]], rows0, gsem0).wait()
            pltpu.async_copy(rows0, acc_sh.at[didx_v.at[j]], ssem0, add=True)
            pltpu.make_async_copy(y_hbm.at[sidx_v.at[j + 1]], rows1, gsem1).wait()
            pltpu.async_copy(rows1, acc_sh.at[didx_v.at[j + 1]], ssem1, add=True)

            @pl.when(p < GRP // 2 - 1)
            def _():
                pltpu.make_async_copy(
                    rows0, acc_sh.at[didx_v.at[j]], ssem0).wait()
                pltpu.async_copy(y_hbm.at[sidx_v.at[j + 2]], rows0, gsem0)
                pltpu.make_async_copy(
                    rows1, acc_sh.at[didx_v.at[j + 1]], ssem1).wait()
                pltpu.async_copy(y_hbm.at[sidx_v.at[j + 3]], rows1, gsem1)

            @pl.when(p == GRP // 2 - 1)
            def _():
                pltpu.make_async_copy(
                    rows0, acc_sh.at[didx_v.at[j]], ssem0).wait()
                pltpu.make_async_copy(
                    rows1, acc_sh.at[didx_v.at[j + 1]], ssem1).wait()


@functools.partial(
    pl.kernel,
    mesh=_mesh(),
    out_type=jax.ShapeDtypeStruct((2, NPAD, D), jnp.float32),
    scratch_types=[
        pltpu.VMEM((GRP, CHUNK), jnp.int32),
        pltpu.VMEM((GRP, CHUNK), jnp.int32),
        pltpu.VMEM_SHARED((NPAD, D), jnp.float32),
        pltpu.SemaphoreType.DMA,
        pltpu.SemaphoreType.DMA,
        pltpu.SemaphoreType.DMA,
        pltpu.SemaphoreType.DMA,
    ],
)
def _aggregate_kernel(y_hbm, sidx_hbm, didx_hbm, zeros_hbm, z_hbm,
                      sidx_v, didx_v, acc_sh, gsem0, gsem1, ssem0, ssem1):
    pl.run_scoped(
        functools.partial(_aggregate_body, y_hbm, sidx_hbm, didx_hbm,
                          zeros_hbm, z_hbm, sidx_v, didx_v, acc_sh,
                          gsem0, gsem1, ssem0, ssem1),
        pltpu.VMEM((CHUNK, D), jnp.float32),
        pltpu.VMEM((CHUNK, D), jnp.float32))


def _aggregate_body(y_hbm, sidx_hbm, didx_hbm, zeros_hbm, z_hbm,
                    sidx_v, didx_v, acc_sh, gsem0, gsem1, ssem0, ssem1,
                    rows0, rows1):
    core = lax.axis_index("c")
    sid = lax.axis_index("s")

    pltpu.sync_copy(zeros_hbm, acc_sh.at[pl.ds(sid * ROWS_PER_TILE, ROWS_PER_TILE)])
    plsc.subcore_barrier()

    args = (y_hbm, sidx_hbm, didx_hbm, acc_sh, sidx_v, didx_v,
            rows0, rows1, gsem0, gsem1, ssem0, ssem1)

    @pl.when(core == 0)
    def _():
        _agg_loop(*args, base0=sid * CPT0, n_groups=CPT0 // GRP)

    @pl.when(core == 1)
    def _():
        _agg_loop(*args, base0=NSUB * CPT0 + sid * CPT1, n_groups=CPT1 // GRP)

    plsc.subcore_barrier()
    pltpu.sync_copy(
        acc_sh.at[pl.ds(sid * ROWS_PER_TILE, ROWS_PER_TILE)],
        z_hbm.at[core, pl.ds(sid * ROWS_PER_TILE, ROWS_PER_TILE)])


def _scale_body(deg2_ref, xp_ref, y_ref, dinv_ref):
    deg = deg2_ref[0, :] + deg2_ref[1, :]
    dinv = jnp.where(deg > 0.0, lax.rsqrt(jnp.maximum(deg, 1.0)), 0.0)
    dinv_ref[...] = dinv[:, None]
    y_ref[...] = xp_ref[...] * dinv[:, None]


def _final_body(x_ref, z_ref, dinv_ref, w_ref, b_ref, o_ref):
    zsum = z_ref[0] + z_ref[1]
    xz = x_ref[...] + dinv_ref[...] * zsum
    r = lax.dot_general(
        xz, w_ref[...], (((1,), (1,)), ((), ())),
        preferred_element_type=jnp.float32, precision=lax.Precision.HIGHEST)
    o_ref[...] = jnp.maximum(r + b_ref[...][None, :], 0.0)


def kernel(x, edge_index, W, b):
    n = x.shape[0]
    n_edges = edge_index.shape[1]
    src = edge_index[0].astype(jnp.int32)
    dst = edge_index[1].astype(jnp.int32)
    pad = jnp.full((E2PAD - 2 * n_edges,), n, jnp.int32)
    sidx = jnp.concatenate([src, dst, pad])
    didx = jnp.concatenate([dst, src, pad])
    sidx_h = sidx.reshape(HROWS, HCHUNK)
    sidx_a = sidx.reshape(IDX_ROWS, CHUNK)
    didx_a = didx.reshape(IDX_ROWS, CHUNK)
    xp = jnp.pad(x, ((0, NPAD - n), (0, 0)))
    zeros1 = jnp.zeros((ROWS_PER_TILE,), jnp.float32)
    zeros2 = jnp.zeros((ROWS_PER_TILE, D), jnp.float32)

    deg2 = _degree_kernel(sidx_h, zeros1)

    y, dinv = pl.pallas_call(
        _scale_body,
        out_shape=[
            jax.ShapeDtypeStruct((NPAD, D), jnp.float32),
            jax.ShapeDtypeStruct((NPAD, 1), jnp.float32),
        ],
    )(deg2, xp)

    z2 = _aggregate_kernel(y, sidx_a, didx_a, zeros2)

    blk = 1000
    out = pl.pallas_call(
        _final_body,
        grid=(n // blk,),
        in_specs=[
            pl.BlockSpec((blk, D), lambda i: (i, 0)),
            pl.BlockSpec((2, blk, D), lambda i: (0, i, 0)),
            pl.BlockSpec((blk, 1), lambda i: (i, 0)),
            pl.BlockSpec((D, D), lambda i: (0, 0)),
            pl.BlockSpec((D,), lambda i: (0,)),
        ],
        out_specs=pl.BlockSpec((blk, D), lambda i: (i, 0)),
        out_shape=jax.ShapeDtypeStruct((n, D), jnp.float32),
    )(x, z2, dinv, W, b)
    return out
